# baseline (speedup 1.0000x reference)
import jax
import jax.numpy as jnp
from jax import lax
from jax.experimental import pallas as pl
from jax.experimental.pallas import tpu as pltpu
from jax.experimental.pallas import tpu_sc as plsc

B = 16384
V = 1000000
DIM = 32
NC, NS = 2, 16
NW = NC * NS
SHARD = 31232
CW = 768
NCH = 42
VMAX = 999168
EPW = 128
EPSTART = 999872
RCAP = 512
WCAP = 256
DUMP = B * DIM
OUTN = B * DIM + 128


def _iota():
  return lax.iota(jnp.int32, 16)


def _scan_table(ut, tail, raw, out, lo, hi,
                chunk_v, list_c, list_p, ring_c, ring_p,
                vals2, oidx2, cnt_ref, sem):
  it16 = _iota()

  def flush():
    w = cnt_ref[2]

    def pad_body(pv, _):
      s = w * 32 + pv * 16
      plsc.store_scatter(oidx2, [(s + it16) >> 7, (s + it16) & 127],
                         jnp.full((16,), DUMP, jnp.int32) + it16)
      return ()

    lax.fori_loop(0, (8192 - w * 32) // 16, pad_body, ())
    for g in range(64):
      pltpu.async_copy(vals2.at[g], out.at[oidx2.at[g]], sem)
    for g in range(64):
      pltpu.make_async_copy(out.at[pl.ds(0, 128)], vals2.at[g], sem).wait()
    cnt_ref[2] = 0

  def drain_ring():
    rw = cnt_ref[1]

    def ex_body(rv, _):
      @pl.when(cnt_ref[2] > WCAP - 16)
      def _():
        flush()
      w = cnt_ref[2]
      ec = ring_c[pl.ds(rv * 16, 16)]
      ep = ring_p[pl.ds(rv * 16, 16)]
      ev = (rv * 16 + it16) < rw
      slots = (w + it16) * 32
      for d in range(DIM):
        g = plsc.load_gather(chunk_v, [jnp.full((16,), d, jnp.int32), ec],
                             mask=ev)
        plsc.store_scatter(vals2, [(slots + d) >> 7, (slots + d) & 127], g,
                           mask=ev)
        plsc.store_scatter(oidx2, [(slots + d) >> 7, (slots + d) & 127],
                           ep * 32 + d, mask=ev)
      cnt_ref[2] = w + jnp.cumsum(ev.astype(jnp.int32))[15]
      return ()

    lax.fori_loop(0, (rw + 15) // 16, ex_body, ())
    cnt_ref[1] = 0

  cnt_ref[0] = 0
  cnt_ref[1] = 0

  def pf_body(g, _):
    n = cnt_ref[0]
    v = raw[pl.ds(g * 16, 16)]
    pos = g * 16 + it16
    m = (v >= lo) & (v < hi)
    c = jnp.cumsum(m.astype(jnp.int32))
    plsc.store_scatter(list_c, [n + c - 1], v, mask=m)
    plsc.store_scatter(list_p, [n + c - 1], pos, mask=m)
    cnt_ref[0] = n + c[15]
    return ()

  lax.fori_loop(0, B // 16, pf_body, ())
  n = cnt_ref[0]

  def win_body(k, _):
    start = jnp.minimum(lo + k * CW, VMAX)
    start = pl.multiple_of(start, 128)
    pltpu.sync_copy(ut.at[:, pl.ds(start, CW)], chunk_v)

    def sc_body(mv, _):
      rw = cnt_ref[1]
      cv = list_c[pl.ds(mv * 16, 16)]
      pv = list_p[pl.ds(mv * 16, 16)]
      m = ((mv * 16 + it16) < n) & (cv >= start) & (cv < start + CW)
      c = jnp.cumsum(m.astype(jnp.int32))
      plsc.store_scatter(ring_c, [rw + c - 1], cv - start, mask=m)
      plsc.store_scatter(ring_p, [rw + c - 1], pv, mask=m)
      cnt_ref[1] = rw + c[15]

      @pl.when(cnt_ref[1] > RCAP - 16)
      def _():
        drain_ring()
      return ()

    lax.fori_loop(0, (n + 15) // 16, sc_body, ())
    drain_ring()
    return ()

  lax.fori_loop(0, NCH, win_body, ())

  pltpu.sync_copy(tail, chunk_v.at[:, pl.ds(0, EPW)])

  def ep_body(mv, _):
    rw = cnt_ref[1]
    cv = list_c[pl.ds(mv * 16, 16)]
    pv = list_p[pl.ds(mv * 16, 16)]
    m = ((mv * 16 + it16) < n) & (cv >= EPSTART)
    c = jnp.cumsum(m.astype(jnp.int32))
    plsc.store_scatter(ring_c, [rw + c - 1], cv - EPSTART, mask=m)
    plsc.store_scatter(ring_p, [rw + c - 1], pv, mask=m)
    cnt_ref[1] = rw + c[15]

    @pl.when(cnt_ref[1] > RCAP - 16)
    def _():
      drain_ring()
    return ()

  lax.fori_loop(0, (n + 15) // 16, ep_body, ())
  drain_ring()
  flush()


def _scan_gather_body(uid_hbm, iid_hbm, ut_hbm, it_hbm, tu_hbm, ti_hbm,
                      ou_hbm, oi_hbm,
                      raw_v, chunk_v, list_c, list_p, ring_c, ring_p,
                      vals2, oidx2, cnt_ref, sem):
  wid = lax.axis_index("c") * NS + lax.axis_index("s")
  lo = wid * SHARD
  hi = jnp.where(wid == NW - 1, V, lo + SHARD)

  pltpu.sync_copy(uid_hbm, raw_v)
  _scan_table(ut_hbm, tu_hbm, raw_v, ou_hbm, lo, hi, chunk_v, list_c, list_p,
              ring_c, ring_p, vals2, oidx2, cnt_ref, sem)
  pltpu.sync_copy(iid_hbm, raw_v)
  _scan_table(it_hbm, ti_hbm, raw_v, oi_hbm, lo, hi, chunk_v, list_c, list_p,
              ring_c, ring_p, vals2, oidx2, cnt_ref, sem)


def _scan_gather(user_id, item_id, ut, it, tu, ti):
  mesh = plsc.VectorSubcoreMesh(core_axis_name="c", subcore_axis_name="s",
                                num_cores=NC, num_subcores=NS)
  k = pl.kernel(
      _scan_gather_body,
      out_type=[jax.ShapeDtypeStruct((OUTN,), jnp.float32),
                jax.ShapeDtypeStruct((OUTN,), jnp.float32)],
      mesh=mesh,
      scratch_types=[
          pltpu.VMEM((B,), jnp.int32),
          pltpu.VMEM((DIM, CW), jnp.float32),
          pltpu.VMEM((B,), jnp.int32),
          pltpu.VMEM((B,), jnp.int32),
          pltpu.VMEM((RCAP,), jnp.int32),
          pltpu.VMEM((RCAP,), jnp.int32),
          pltpu.VMEM((64, 128), jnp.float32),
          pltpu.VMEM((64, 128), jnp.int32),
          pltpu.SMEM((4,), jnp.int32),
          pltpu.SemaphoreType.DMA,
      ],
      name="ncf_sc_scan_gather",
      compiler_params=pltpu.CompilerParams(use_tc_tiling_on_sc=True,
                                           needs_layout_passes=False),
  )
  return k(user_id, item_id, ut, it, tu, ti)


_SMALL_SIZES = (50, 20, 100, 2)


def _mlp_body(xu, xi, gid, lid, aid, gnd,
              genre_t, lang_t, age_t, gender_t,
              w1u, w1i, w1g, w1l, w1a, w1n,
              b1, w2t, b2, w3t, b3, out):
  bm = xu.shape[0]
  acc = jnp.dot(xu[...], w1u[...], preferred_element_type=jnp.float32)
  acc = acc + jnp.dot(xi[...], w1i[...], preferred_element_type=jnp.float32)
  ids = (gid, lid, aid, gnd)
  tabs = (genre_t, lang_t, age_t, gender_t)
  ws = (w1g, w1l, w1a, w1n)
  for t in range(4):
    ncat = _SMALL_SIZES[t]
    proj = jnp.dot(tabs[t][...], ws[t][...],
                   preferred_element_type=jnp.float32)
    cats = jax.lax.broadcasted_iota(jnp.int32, (bm, ncat), 1)
    onehot = (ids[t][...] == cats).astype(jnp.float32)
    acc = acc + jnp.dot(onehot, proj, preferred_element_type=jnp.float32)
  h1 = jnp.maximum(acc + b1[...], 0.0)
  h2 = jnp.maximum(jnp.dot(h1, w2t[...], preferred_element_type=jnp.float32)
                   + b2[...], 0.0)
  z = jnp.dot(h2, w3t[...], preferred_element_type=jnp.float32) + b3[...]
  out[...] = 1.0 / (1.0 + jnp.exp(-z))


def _mlp(xu, xi, gid, lid, aid, gnd, genre_emb, lang_emb, age_emb, gender_emb,
         w1ts, b1, w2t, b2, w3t, b3):
  bm = 2048
  grid = (B // bm,)
  x_spec = pl.BlockSpec((bm, DIM), lambda i: (i, 0))
  id_spec = pl.BlockSpec((bm, 1), lambda i: (i, 0))
  full = lambda shape: pl.BlockSpec(shape, lambda i: (0, 0))
  in_specs = ([x_spec, x_spec] + [id_spec] * 4
              + [full((n, DIM)) for n in _SMALL_SIZES]
              + [full((DIM, 64))] * 6
              + [full((1, 64)), full((64, 32)), full((1, 32)),
                 full((32, 1)), full((1, 1))])
  return pl.pallas_call(
      _mlp_body,
      grid=grid,
      in_specs=in_specs,
      out_specs=pl.BlockSpec((bm, 1), lambda i: (i, 0)),
      out_shape=jax.ShapeDtypeStruct((B, 1), jnp.float32),
  )(xu, xi, gid.reshape(B, 1), lid.reshape(B, 1),
    aid.reshape(B, 1), gnd.reshape(B, 1),
    genre_emb, lang_emb, age_emb, gender_emb, *w1ts, b1, w2t, b2, w3t, b3)


def kernel(user_id, item_id, genre_id, language_id, age, gender,
           user_emb, item_emb, genre_emb, lang_emb, age_emb, gender_emb,
           W1, b1, W2, b2, W3, b3):
  ou, oi = _scan_gather(user_id.astype(jnp.int32), item_id.astype(jnp.int32),
                        user_emb.T, item_emb.T,
                        user_emb[EPSTART:].T, item_emb[EPSTART:].T)
  xu = ou[:B * DIM].reshape(B, DIM)
  xi = oi[:B * DIM].reshape(B, DIM)

  w1t = W1.T
  w1ts = [w1t[DIM * t:DIM * (t + 1)] for t in range(6)]
  return _mlp(xu, xi, genre_id.astype(jnp.int32),
              language_id.astype(jnp.int32),
              age.astype(jnp.int32), gender.astype(jnp.int32),
              genre_emb, lang_emb, age_emb, gender_emb,
              w1ts, b1.reshape(1, 64), W2.T, b2.reshape(1, 32),
              W3.T, b3.reshape(1, 1))


# speedup vs baseline: 41.3340x; 41.3340x over previous
import jax
import jax.numpy as jnp
from jax import lax
from jax.experimental import pallas as pl
from jax.experimental.pallas import tpu as pltpu
from jax.experimental.pallas import tpu_sc as plsc

B = 16384
DIM = 32
NC, NS = 2, 16
NW = NC * NS
BPW = B // NW


def _gather_body(uid_hbm, iid_hbm, ut_hbm, it_hbm, ou_hbm, oi_hbm,
                 idx_u, idx_i, rows_u, rows_i, sem_u, sem_i):
  wid = lax.axis_index("c") * NS + lax.axis_index("s")
  base = wid * BPW
  pltpu.sync_copy(uid_hbm.at[pl.ds(base, BPW)], idx_u)
  pltpu.sync_copy(iid_hbm.at[pl.ds(base, BPW)], idx_i)
  cu = pltpu.async_copy(ut_hbm.at[idx_u], rows_u, sem_u)
  ci = pltpu.async_copy(it_hbm.at[idx_i], rows_i, sem_i)
  cu.wait()
  pltpu.sync_copy(rows_u, ou_hbm.at[pl.ds(base, BPW)])
  ci.wait()
  pltpu.sync_copy(rows_i, oi_hbm.at[pl.ds(base, BPW)])


def _gather(user_id, item_id, ut, it):
  mesh = plsc.VectorSubcoreMesh(core_axis_name="c", subcore_axis_name="s",
                                num_cores=NC, num_subcores=NS)
  k = pl.kernel(
      _gather_body,
      out_type=[jax.ShapeDtypeStruct((B, DIM), jnp.float32),
                jax.ShapeDtypeStruct((B, DIM), jnp.float32)],
      mesh=mesh,
      scratch_types=[
          pltpu.VMEM((BPW,), jnp.int32),
          pltpu.VMEM((BPW,), jnp.int32),
          pltpu.VMEM((BPW, DIM), jnp.float32),
          pltpu.VMEM((BPW, DIM), jnp.float32),
          pltpu.SemaphoreType.DMA,
          pltpu.SemaphoreType.DMA,
      ],
      name="ncf_sc_gather",
      compiler_params=pltpu.CompilerParams(use_tc_tiling_on_sc=False),
  )
  return k(user_id, item_id, ut, it)


_SMALL_SIZES = (50, 20, 100, 2)


def _mlp_body(xu, xi, gid, lid, aid, gnd,
              genre_t, lang_t, age_t, gender_t,
              w1u, w1i, w1g, w1l, w1a, w1n,
              b1, w2t, b2, w3t, b3, out):
  bm = xu.shape[0]
  acc = jnp.dot(xu[...], w1u[...], preferred_element_type=jnp.float32)
  acc = acc + jnp.dot(xi[...], w1i[...], preferred_element_type=jnp.float32)
  ids = (gid, lid, aid, gnd)
  tabs = (genre_t, lang_t, age_t, gender_t)
  ws = (w1g, w1l, w1a, w1n)
  for t in range(4):
    ncat = _SMALL_SIZES[t]
    proj = jnp.dot(tabs[t][...], ws[t][...],
                   preferred_element_type=jnp.float32)
    cats = jax.lax.broadcasted_iota(jnp.int32, (bm, ncat), 1)
    onehot = (ids[t][...] == cats).astype(jnp.float32)
    acc = acc + jnp.dot(onehot, proj, preferred_element_type=jnp.float32)
  h1 = jnp.maximum(acc + b1[...], 0.0)
  h2 = jnp.maximum(jnp.dot(h1, w2t[...], preferred_element_type=jnp.float32)
                   + b2[...], 0.0)
  z = jnp.dot(h2, w3t[...], preferred_element_type=jnp.float32) + b3[...]
  out[...] = 1.0 / (1.0 + jnp.exp(-z))


def _mlp(xu, xi, gid, lid, aid, gnd, genre_emb, lang_emb, age_emb, gender_emb,
         w1ts, b1, w2t, b2, w3t, b3):
  bm = 2048
  grid = (B // bm,)
  x_spec = pl.BlockSpec((bm, DIM), lambda i: (i, 0))
  id_spec = pl.BlockSpec((bm, 1), lambda i: (i, 0))
  full = lambda shape: pl.BlockSpec(shape, lambda i: (0, 0))
  in_specs = ([x_spec, x_spec] + [id_spec] * 4
              + [full((n, DIM)) for n in _SMALL_SIZES]
              + [full((DIM, 64))] * 6
              + [full((1, 64)), full((64, 32)), full((1, 32)),
                 full((32, 1)), full((1, 1))])
  return pl.pallas_call(
      _mlp_body,
      grid=grid,
      in_specs=in_specs,
      out_specs=pl.BlockSpec((bm, 1), lambda i: (i, 0)),
      out_shape=jax.ShapeDtypeStruct((B, 1), jnp.float32),
  )(xu, xi, gid.reshape(B, 1), lid.reshape(B, 1),
    aid.reshape(B, 1), gnd.reshape(B, 1),
    genre_emb, lang_emb, age_emb, gender_emb, *w1ts, b1, w2t, b2, w3t, b3)


def kernel(user_id, item_id, genre_id, language_id, age, gender,
           user_emb, item_emb, genre_emb, lang_emb, age_emb, gender_emb,
           W1, b1, W2, b2, W3, b3):
  xu, xi = _gather(user_id.astype(jnp.int32), item_id.astype(jnp.int32),
                   user_emb, item_emb)

  w1t = W1.T
  w1ts = [w1t[DIM * t:DIM * (t + 1)] for t in range(6)]
  return _mlp(xu, xi, genre_id.astype(jnp.int32),
              language_id.astype(jnp.int32),
              age.astype(jnp.int32), gender.astype(jnp.int32),
              genre_emb, lang_emb, age_emb, gender_emb,
              w1ts, b1.reshape(1, 64), W2.T, b2.reshape(1, 32),
              W3.T, b3.reshape(1, 1))
